# Initial kernel scaffold; baseline (speedup 1.0000x reference)
import jax
import jax.numpy as jnp
from jax.experimental import pallas as pl


def kernel(x, knowledge_embeddings, Wq, bq, Wk, bk, Wv, bv, Wo, bo):
    raise NotImplementedError("write your pallas kernel here")


# speedup vs baseline: 45.0977x; 45.0977x over previous
import functools

import jax
import jax.numpy as jnp
import numpy as np
from jax.experimental import pallas as pl
from jax.experimental.pallas import tpu as pltpu

_H = 16
_K = 16
_SBLK = 512


def _nt(a, b):
    return jax.lax.dot_general(a, b, (((1,), (1,)), ((), ())),
                               preferred_element_type=jnp.float32)


def _kv_proj_kernel(kb_ref, Wk_ref, bk_ref, Wv_ref, bv_ref, k_ref, v_ref):
    kb = kb_ref[0]
    k_ref[0] = _nt(kb, Wk_ref[...]) + bk_ref[...]
    v_ref[0] = _nt(kb, Wv_ref[...]) + bv_ref[...]


def _attn_kernel(x_ref, k_ref, v_ref, Wq_ref, bq_ref, Wo_ref, bo_ref,
                 out_ref, ctx_ref, *, inv_scale):
    x = x_ref[0]
    q = _nt(x, Wq_ref[...]) + bq_ref[...]
    kk = k_ref[0]
    vv = v_ref[0]
    hd = q.shape[1] // _H
    for h in range(_H):
        sl = slice(h * hd, (h + 1) * hd)
        s = _nt(q[:, sl], kk[:, sl]) * inv_scale
        w = s
        t = None
        for _ in range(_K):
            t = jnp.max(w, axis=1, keepdims=True)
            w = jnp.where(w >= t, -jnp.inf, w)
        p = jnp.where(s >= t, jnp.exp(s - t), 0.0)
        denom = jnp.sum(p, axis=1, keepdims=True)
        ctx_ref[:, sl] = jnp.dot(p, vv[:, sl],
                                 preferred_element_type=jnp.float32) / denom
    out_ref[0] = _nt(ctx_ref[...], Wo_ref[...]) + bo_ref[...]


@jax.jit
def kernel(x, knowledge_embeddings, Wq, bq, Wk, bk, Wv, bv, Wo, bo):
    B, S, D = x.shape
    N = knowledge_embeddings.shape[1]
    inv_scale = float(1.0 / np.sqrt(D // _H))

    full2 = pl.BlockSpec((D, D), lambda *_: (0, 0))
    full1 = pl.BlockSpec((1, D), lambda *_: (0, 0))
    bnd = pl.BlockSpec((1, N, D), lambda b, *_: (b, 0, 0))

    k_proj, v_proj = pl.pallas_call(
        _kv_proj_kernel,
        grid=(B,),
        in_specs=[bnd, full2, full1, full2, full1],
        out_specs=[bnd, bnd],
        out_shape=[jax.ShapeDtypeStruct((B, N, D), jnp.float32)] * 2,
    )(knowledge_embeddings, Wk, bk.reshape(1, D), Wv, bv.reshape(1, D))

    out = pl.pallas_call(
        functools.partial(_attn_kernel, inv_scale=inv_scale),
        grid=(B, S // _SBLK),
        in_specs=[
            pl.BlockSpec((1, _SBLK, D), lambda b, s: (b, s, 0)),
            bnd, bnd, full2, full1, full2, full1,
        ],
        out_specs=pl.BlockSpec((1, _SBLK, D), lambda b, s: (b, s, 0)),
        out_shape=jax.ShapeDtypeStruct((B, S, D), jnp.float32),
        scratch_shapes=[pltpu.VMEM((_SBLK, D), jnp.float32)],
    )(x, k_proj, v_proj, Wq, bq.reshape(1, D), Wo, bo.reshape(1, D))
    return out
